# Initial kernel scaffold; baseline (speedup 1.0000x reference)
import jax
import jax.numpy as jnp
from jax.experimental import pallas as pl


def kernel(ligand_atom, ligand_pos, ligand_pad_mask, params):
    raise NotImplementedError("write your pallas kernel here")


# speedup vs baseline: 13.1068x; 13.1068x over previous
import jax
import jax.numpy as jnp
from jax import lax
from jax.experimental import pallas as pl

_N = 48
_IN = 16
_OUT = 4
_H = 128
_NL = 6
_CUT = 6.0
_DELTA = _CUT / (_H - 1)
_COEFF = -0.5 / (_DELTA * _DELTA)
_LOG2 = 0.6931471805599453


def _ssp(x):
    return jnp.maximum(x, 0.0) + jnp.log1p(jnp.exp(-jnp.abs(x))) - _LOG2


def _dot(a, b):
    return jnp.dot(a, b, preferred_element_type=jnp.float32)


def _fwd_kernel(atom_ref, pos_ref, mask_ref, *refs):
    (enc_emb_W, enc_emb_b,
     e_lin1W, e_nn0W, e_nn0b, e_nn1W, e_nn1b, e_lin2W, e_lin2b, e_linW, e_linb,
     dec_emb_W, dec_emb_b,
     d_lin1W, d_nn0W, d_nn0b, d_nn1W, d_nn1b, d_lin2W, d_lin2b, d_linW, d_linb,
     f1mW, f1mb, f2mW, f2mb, f3mW, f3mb,
     f1vW, f1vb, f2vW, f2vb, f3vW, f3vb,
     o1W, o1b, o2W, o2b,
     recon_ref, kl_ref) = refs

    maskc = mask_ref[0]
    atom = atom_ref[0] * maskc
    pos = pos_ref[0] * maskc

    px = pos[:, 0:1]
    py = pos[:, 1:2]
    pz = pos[:, 2:3]
    dx = px - px.reshape(1, _N)
    dy = py - py.reshape(1, _N)
    dz = pz - pz.reshape(1, _N)
    el = jnp.sqrt(dx * dx + dy * dy + dz * dz)

    row = lax.broadcasted_iota(jnp.int32, (_N, _N), 0)
    col = lax.broadcasted_iota(jnp.int32, (_N, _N), 1)
    keep = (el <= _CUT) & (row != col)
    cenv = jnp.where(keep, 0.5 * (jnp.cos(el * (jnp.pi / _CUT)) + 1.0), 0.0)
    c3 = cenv[:, :, None]

    offs = lax.broadcasted_iota(jnp.int32, (1, 1, _H), 2).astype(jnp.float32) * _DELTA
    diff = el[:, :, None] - offs
    ea = jnp.exp(_COEFF * diff * diff).reshape(_N * _N, _H)

    def schnet(h, lin1W, nn0W, nn0b, nn1W, nn1b, lin2W, lin2b, linW, linb):
        for b in range(_NL):
            t = _ssp(_dot(ea, nn0W[b]) + nn0b[b][None, :])
            t = _dot(t, nn1W[b]) + nn1b[b][None, :]
            wf = t.reshape(_N, _N, _H) * c3
            hx = _dot(h, lin1W[b])
            agg = jnp.sum(hx[:, None, :] * wf, axis=0)
            hh = _ssp(_dot(agg, lin2W[b]) + lin2b[b][None, :])
            hh = _dot(hh, linW[b]) + linb[b][None, :]
            h = h + hh
        return h

    h = _dot(atom, enc_emb_W[...]) + enc_emb_b[...]
    h = schnet(h, e_lin1W, e_nn0W, e_nn0b, e_nn1W, e_nn1b,
               e_lin2W, e_lin2b, e_linW, e_linb)

    m = jnp.maximum(_dot(h, f1mW[...]) + f1mb[...], 0.0)
    m = jnp.maximum(_dot(m, f2mW[...]) + f2mb[...], 0.0)
    m = _dot(m, f3mW[...]) + f3mb[...]
    v = jnp.maximum(_dot(h, f1vW[...]) + f1vb[...], 0.0)
    v = jnp.maximum(_dot(v, f2vW[...]) + f2vb[...], 0.0)
    v = _dot(v, f3vW[...]) + f3vb[...]

    klp = 0.5 * jnp.sum(jnp.exp(v) + m * m - 1.0 - v)

    h2 = _dot(m, dec_emb_W[...]) + dec_emb_b[...]
    h2 = schnet(h2, d_lin1W, d_nn0W, d_nn0b, d_nn1W, d_nn1b,
                d_lin2W, d_lin2b, d_linW, d_linb)

    f = jnp.maximum(_dot(h2, o1W[...]) + o1b[...], 0.0)
    f = _dot(f, o2W[...]) + o2b[...]

    recon_ref[0] = f
    kl_ref[...] = jnp.broadcast_to(klp, (1, 1, _H))


def kernel(ligand_atom, ligand_pos, ligand_pad_mask, params):
    bs = ligand_atom.shape[0]
    p = params
    enc = p['enc_blocks']
    dec = p['dec_blocks']

    def stk(blocks, k):
        return jnp.stack([blk[k] for blk in blocks])

    def rb(b):
        return b.reshape(1, -1)

    wlist = [
        p['enc_emb_W'], rb(p['enc_emb_b']),
        stk(enc, 'lin1_W'), stk(enc, 'nn0_W'), stk(enc, 'nn0_b'),
        stk(enc, 'nn1_W'), stk(enc, 'nn1_b'), stk(enc, 'lin2_W'),
        stk(enc, 'lin2_b'), stk(enc, 'lin_W'), stk(enc, 'lin_b'),
        p['dec_emb_W'], rb(p['dec_emb_b']),
        stk(dec, 'lin1_W'), stk(dec, 'nn0_W'), stk(dec, 'nn0_b'),
        stk(dec, 'nn1_W'), stk(dec, 'nn1_b'), stk(dec, 'lin2_W'),
        stk(dec, 'lin2_b'), stk(dec, 'lin_W'), stk(dec, 'lin_b'),
        p['fc1_m_W'], rb(p['fc1_m_b']), p['fc2_m_W'], rb(p['fc2_m_b']),
        p['fc3_m_W'], rb(p['fc3_m_b']),
        p['fc1_v_W'], rb(p['fc1_v_b']), p['fc2_v_W'], rb(p['fc2_v_b']),
        p['fc3_v_W'], rb(p['fc3_v_b']),
        p['out1_W'], rb(p['out1_b']), p['out2_W'], rb(p['out2_b']),
    ]

    mask_r = ligand_pad_mask.reshape(bs, _N, 1)

    def const_spec(w):
        nd = w.ndim
        return pl.BlockSpec(w.shape, (lambda *_: (0,) * nd))

    in_specs = [
        pl.BlockSpec((1, _N, _IN), lambda i: (i, 0, 0)),
        pl.BlockSpec((1, _N, 3), lambda i: (i, 0, 0)),
        pl.BlockSpec((1, _N, 1), lambda i: (i, 0, 0)),
    ] + [const_spec(w) for w in wlist]

    out_specs = [
        pl.BlockSpec((1, _N, _IN), lambda i: (i, 0, 0)),
        pl.BlockSpec((1, 1, _H), lambda i: (i, 0, 0)),
    ]
    out_shape = [
        jax.ShapeDtypeStruct((bs, _N, _IN), jnp.float32),
        jax.ShapeDtypeStruct((bs, 1, _H), jnp.float32),
    ]

    recon, klp = pl.pallas_call(
        _fwd_kernel,
        grid=(bs,),
        in_specs=in_specs,
        out_specs=out_specs,
        out_shape=out_shape,
    )(ligand_atom, ligand_pos, mask_r, *wlist)

    kl = jnp.sum(klp[:, 0, 0])
    return recon, kl


# speedup vs baseline: 19.6569x; 1.4998x over previous
import jax
import jax.numpy as jnp
from jax import lax
from jax.experimental import pallas as pl

_N = 48
_IN = 16
_OUT = 4
_H = 128
_NL = 6
_CUT = 6.0
_DELTA = _CUT / (_H - 1)
_COEFF = -0.5 / (_DELTA * _DELTA)
_LOG2 = 0.6931471805599453


def _sp(x):
    return jnp.log(1.0 + jnp.exp(x))


def _dot(a, b):
    return jnp.dot(a, b, preferred_element_type=jnp.float32)


def _fwd_kernel(atom_ref, pos_ref, mask_ref, *refs):
    (enc_emb_W, enc_emb_b,
     e_lin1W, e_nn0W, e_nn0b, e_nn1W, e_nn1b, e_lin2W, e_lin2b, e_linW, e_linb,
     dec_emb_W, dec_emb_b,
     d_lin1W, d_nn0W, d_nn0b, d_nn1W, d_nn1b, d_lin2W, d_lin2b, d_linW, d_linb,
     f1mW, f1mb, f2mW, f2mb, f3mW, f3mb,
     f1vW, f1vb, f2vW, f2vb, f3vW, f3vb,
     o1W, o1b, o2W, o2b,
     recon_ref, kl_ref) = refs

    maskc = mask_ref[0]
    atom = atom_ref[0] * maskc
    pos = pos_ref[0] * maskc

    px = pos[:, 0:1]
    py = pos[:, 1:2]
    pz = pos[:, 2:3]
    dx = px - px.reshape(1, _N)
    dy = py - py.reshape(1, _N)
    dz = pz - pz.reshape(1, _N)
    el = jnp.sqrt(dx * dx + dy * dy + dz * dz)

    row = lax.broadcasted_iota(jnp.int32, (_N, _N), 0)
    col = lax.broadcasted_iota(jnp.int32, (_N, _N), 1)
    keep = (el <= _CUT) & (row != col)
    cenv = jnp.where(keep, 0.5 * (jnp.cos(el * (jnp.pi / _CUT)) + 1.0), 0.0)
    c3 = cenv[:, :, None]

    offs = lax.broadcasted_iota(jnp.int32, (1, 1, _H), 2).astype(jnp.float32) * _DELTA
    diff = el[:, :, None] - offs
    ea = jnp.exp(_COEFF * diff * diff).reshape(_N * _N, _H)

    def schnet(h, lin1W, nn0W, nn0b, nn1W, nn1b, lin2W, lin2b, linW, linb):
        for b in range(_NL):
            t = _sp(_dot(ea, nn0W[b]) + nn0b[b][None, :])
            t = _dot(t, nn1W[b]) + nn1b[b][None, :]
            wf = t.reshape(_N, _N, _H) * c3
            hx = _dot(h, lin1W[b])
            agg = jnp.sum(hx[:, None, :] * wf, axis=0)
            hh = _sp(_dot(agg, lin2W[b]) + lin2b[b][None, :])
            hh = _dot(hh, linW[b]) + linb[b][None, :]
            h = h + hh
        return h

    h = _dot(atom, enc_emb_W[...]) + enc_emb_b[...]
    h = schnet(h, e_lin1W, e_nn0W, e_nn0b, e_nn1W, e_nn1b,
               e_lin2W, e_lin2b, e_linW, e_linb)

    m = jnp.maximum(_dot(h, f1mW[...]) + f1mb[...], 0.0)
    m = jnp.maximum(_dot(m, f2mW[...]) + f2mb[...], 0.0)
    m = _dot(m, f3mW[...]) + f3mb[...]
    v = jnp.maximum(_dot(h, f1vW[...]) + f1vb[...], 0.0)
    v = jnp.maximum(_dot(v, f2vW[...]) + f2vb[...], 0.0)
    v = _dot(v, f3vW[...]) + f3vb[...]

    klp = 0.5 * jnp.sum(jnp.exp(v) + m * m - 1.0 - v)

    h2 = _dot(m, dec_emb_W[...]) + dec_emb_b[...]
    h2 = schnet(h2, d_lin1W, d_nn0W, d_nn0b, d_nn1W, d_nn1b,
                d_lin2W, d_lin2b, d_linW, d_linb)

    f = jnp.maximum(_dot(h2, o1W[...]) + o1b[...], 0.0)
    f = _dot(f, o2W[...]) + o2b[...]

    recon_ref[0] = f
    kl_ref[...] = jnp.broadcast_to(klp, (1, 1, _H))


def kernel(ligand_atom, ligand_pos, ligand_pad_mask, params):
    bs = ligand_atom.shape[0]
    p = params
    enc = p['enc_blocks']
    dec = p['dec_blocks']

    def stk(blocks, k):
        return jnp.stack([blk[k] for blk in blocks])

    def rb(b):
        return b.reshape(1, -1)

    def shift_b(blocks, bk, wk):
        return stk(blocks, bk) - _LOG2 * jnp.sum(stk(blocks, wk), axis=1)

    wlist = [
        p['enc_emb_W'], rb(p['enc_emb_b']),
        stk(enc, 'lin1_W'), stk(enc, 'nn0_W'), stk(enc, 'nn0_b'),
        stk(enc, 'nn1_W'), shift_b(enc, 'nn1_b', 'nn1_W'), stk(enc, 'lin2_W'),
        stk(enc, 'lin2_b'), stk(enc, 'lin_W'), shift_b(enc, 'lin_b', 'lin_W'),
        p['dec_emb_W'], rb(p['dec_emb_b']),
        stk(dec, 'lin1_W'), stk(dec, 'nn0_W'), stk(dec, 'nn0_b'),
        stk(dec, 'nn1_W'), shift_b(dec, 'nn1_b', 'nn1_W'), stk(dec, 'lin2_W'),
        stk(dec, 'lin2_b'), stk(dec, 'lin_W'), shift_b(dec, 'lin_b', 'lin_W'),
        p['fc1_m_W'], rb(p['fc1_m_b']), p['fc2_m_W'], rb(p['fc2_m_b']),
        p['fc3_m_W'], rb(p['fc3_m_b']),
        p['fc1_v_W'], rb(p['fc1_v_b']), p['fc2_v_W'], rb(p['fc2_v_b']),
        p['fc3_v_W'], rb(p['fc3_v_b']),
        p['out1_W'], rb(p['out1_b']), p['out2_W'], rb(p['out2_b']),
    ]

    mask_r = ligand_pad_mask.reshape(bs, _N, 1)

    def const_spec(w):
        nd = w.ndim
        return pl.BlockSpec(w.shape, (lambda *_: (0,) * nd))

    in_specs = [
        pl.BlockSpec((1, _N, _IN), lambda i: (i, 0, 0)),
        pl.BlockSpec((1, _N, 3), lambda i: (i, 0, 0)),
        pl.BlockSpec((1, _N, 1), lambda i: (i, 0, 0)),
    ] + [const_spec(w) for w in wlist]

    out_specs = [
        pl.BlockSpec((1, _N, _IN), lambda i: (i, 0, 0)),
        pl.BlockSpec((1, 1, _H), lambda i: (i, 0, 0)),
    ]
    out_shape = [
        jax.ShapeDtypeStruct((bs, _N, _IN), jnp.float32),
        jax.ShapeDtypeStruct((bs, 1, _H), jnp.float32),
    ]

    recon, klp = pl.pallas_call(
        _fwd_kernel,
        grid=(bs,),
        in_specs=in_specs,
        out_specs=out_specs,
        out_shape=out_shape,
    )(ligand_atom, ligand_pos, mask_r, *wlist)

    kl = jnp.sum(klp[:, 0, 0])
    return recon, kl
